# baseline (speedup 1.0000x reference)
import functools

import jax
import jax.numpy as jnp
from jax import lax
from jax.experimental import pallas as pl
from jax.experimental.pallas import tpu as pltpu
from jax.experimental.pallas import tpu_sc as plsc

B, L, D, V = 32, 4096, 256, 5
N = B * L

_SC_INFO = plsc.get_sparse_core_info()
NC = _SC_INFO.num_cores
NS = _SC_INFO.num_subcores
NW = NC * NS
TOK_PER_W = N // NW
CT = 128
NCHUNK = TOK_PER_W // CT


def _ln_body(tab_ref, gamma_ref, beta_ref, out_ref):
    tab = tab_ref[...]
    mean = jnp.mean(tab, axis=1, keepdims=True)
    var = jnp.mean((tab - mean) ** 2, axis=1, keepdims=True)
    ntab = (tab - mean) * jax.lax.rsqrt(var + 1e-12)
    out_ref[...] = ntab * gamma_ref[...] + beta_ref[...]


def _normed_table(emb_table, ln_gamma, ln_beta):
    return pl.pallas_call(
        _ln_body,
        out_shape=jax.ShapeDtypeStruct((V, D), jnp.float32),
    )(emb_table, ln_gamma.reshape(1, D), ln_beta.reshape(1, D))


@functools.partial(
    pl.kernel,
    out_type=jax.ShapeDtypeStruct((N * D,), jnp.float32),
    mesh=plsc.VectorSubcoreMesh(core_axis_name="c", subcore_axis_name="s"),
    scratch_types=[
        pltpu.VMEM((V * D,), jnp.float32),
        pltpu.VMEM((TOK_PER_W,), jnp.int32),
        pltpu.VMEM((CT * D,), jnp.float32),
        pltpu.VMEM((CT * D,), jnp.float32),
        pltpu.SemaphoreType.DMA,
        pltpu.SemaphoreType.DMA,
    ],
)
def _sc_expand(ntab_hbm, ids_hbm, out_hbm, ntab_v, idx_all, rows0, rows1,
               osem0, osem1):
    wid = lax.axis_index("s") * NC + lax.axis_index("c")
    base = wid * TOK_PER_W

    pltpu.sync_copy(ntab_hbm, ntab_v)
    pltpu.sync_copy(ids_hbm.at[pl.ds(base, TOK_PER_W)], idx_all)

    def expand(g, rows, osem):
        def grp(h, carry):
            ids16 = idx_all[pl.ds(g * CT + h * 16, 16)]
            for m in range(16):
                row = pl.multiple_of(ids16[m] * D, 8)
                dst = pl.multiple_of((h * 16 + m) * D, 8)
                for k in range(D // 16):
                    rows[pl.ds(dst + k * 16, 16)] = (
                        ntab_v[pl.ds(row + k * 16, 16)])
            return carry

        lax.fori_loop(0, CT // 16, grp, 0)
        pltpu.make_async_copy(
            rows, out_hbm.at[pl.ds((base + g * CT) * D, CT * D)], osem
        ).start()

    def out_wait(g, rows, osem):
        pltpu.make_async_copy(
            rows, out_hbm.at[pl.ds((base + g * CT) * D, CT * D)], osem
        ).wait()

    npairs = NCHUNK // 2

    def pair(g2, carry):
        g = g2 * 2

        @pl.when(g2 > 0)
        def _():
            out_wait(g - 2, rows0, osem0)

        expand(g, rows0, osem0)

        @pl.when(g2 > 0)
        def _():
            out_wait(g - 1, rows1, osem1)

        expand(g + 1, rows1, osem1)
        return carry

    lax.fori_loop(0, npairs, pair, 0)
    out_wait(NCHUNK - 2, rows0, osem0)
    out_wait(NCHUNK - 1, rows1, osem1)


@functools.partial(jax.jit, static_argnames=())
def kernel(snp_ids, is_padding, emb_table, ln_gamma, ln_beta):
    ntab = _normed_table(emb_table, ln_gamma, ln_beta)
    out = _sc_expand(ntab.reshape(V * D), snp_ids.reshape(N))
    return out.reshape(B, L, D), is_padding
